# baseline (speedup 1.0000x reference)
import functools

import jax
import jax.numpy as jnp
from jax.experimental import pallas as pl
from jax.experimental.pallas import tpu as pltpu

_ROWS = 400


def _body(x_ref, adj_ref, cache_ref, w1_ref, b1_ref, w2_ref, b2_ref, watt_ref,
          batt_ref, wcls_ref, bcls_ref, out_ref,
          hba_ref, hb_ref, m_ref, d_ref, g_ref, *, nblk, r, h):
    t = pl.program_id(0)
    c = nblk - 2

    @pl.when(t == 0)
    def _init():
        hba_ref[:, 2 * h:3 * h] = (
            jnp.dot(x_ref[...], w1_ref[...], preferred_element_type=jnp.float32)
            + b1_ref[...]
        )
        m_ref[0, 0] = -jnp.inf
        d_ref[0, 0] = 0.0
        g_ref[...] = jnp.zeros_like(g_ref)

    @pl.when(t < nblk)
    def _pass1():
        a = hba_ref[:, 2 * h:3 * h]

        @pl.when(t != c)
        def _stream():
            hb_ref[...] = jnp.dot(adj_ref[...], a,
                                  preferred_element_type=jnp.float32)

        @pl.when(t == c)
        def _cached():
            hb_ref[...] = jnp.dot(cache_ref[...], a,
                                  preferred_element_type=jnp.float32)

        h1 = jnp.maximum(hb_ref[...], 0.0)
        hba_ref[pl.ds(t * r, r), 0:h] = h1
        hba_ref[pl.ds(t * r, r), h:2 * h] = (
            jnp.dot(h1, w2_ref[...], preferred_element_type=jnp.float32)
            + b2_ref[...]
        )

    @pl.when(t >= nblk)
    def _pass2():
        l = 2 * nblk - 1 - t
        bm = hba_ref[:, h:2 * h]

        @pl.when(l != c)
        def _stream():
            hb_ref[...] = jnp.dot(adj_ref[...], bm,
                                  preferred_element_type=jnp.float32)

        @pl.when(l == c)
        def _cached():
            hb_ref[...] = jnp.dot(cache_ref[...], bm,
                                  preferred_element_type=jnp.float32)

        h2 = jnp.maximum(hb_ref[...], 0.0)
        hrow = hba_ref[pl.ds(l * r, r), 0:h] + h2
        s = (
            jnp.dot(hrow, watt_ref[...], preferred_element_type=jnp.float32)
            + batt_ref[0, 0]
        )

        m_old = m_ref[0, 0]
        m_new = jnp.maximum(m_old, jnp.max(s))
        scale = jnp.exp(m_old - m_new)
        e = jnp.exp(s - m_new)
        d_ref[0, 0] = d_ref[0, 0] * scale + jnp.sum(e)
        g_ref[...] = g_ref[...] * scale + jnp.sum(e * hrow, axis=0,
                                                  keepdims=True)
        m_ref[0, 0] = m_new

        @pl.when(t == 2 * nblk - 1)
        def _fini():
            g = g_ref[...] / d_ref[0, 0]
            out_ref[...] = (
                jnp.dot(g, wcls_ref[...], preferred_element_type=jnp.float32)
                + bcls_ref[...]
            )


def kernel(x, adj, W1, b1, W2, b2, Watt, batt, Wcls, bcls):
    N, DIN = x.shape
    H = W1.shape[1]
    C = Wcls.shape[1]
    R = _ROWS
    nblk = N // R
    c = nblk - 2
    f32 = jnp.float32

    def adj_map(t):
        l = 2 * nblk - 1 - t
        p1 = jnp.where(t == c, c - 1, t)
        p2 = jnp.where(l == c, c + 1, l)
        return (jnp.where(t < nblk, p1, p2), 0)

    const = lambda t: (0, 0)
    cache_map = lambda t: (c, 0)
    out = pl.pallas_call(
        functools.partial(_body, nblk=nblk, r=R, h=H),
        grid=(2 * nblk,),
        in_specs=[
            pl.BlockSpec((N, DIN), const),
            pl.BlockSpec((R, N), adj_map),
            pl.BlockSpec((R, N), cache_map),
            pl.BlockSpec((DIN, H), const),
            pl.BlockSpec((1, H), const),
            pl.BlockSpec((H, H), const),
            pl.BlockSpec((1, H), const),
            pl.BlockSpec((H, 1), const),
            pl.BlockSpec((1, 1), const),
            pl.BlockSpec((H, C), const),
            pl.BlockSpec((1, C), const),
        ],
        out_specs=pl.BlockSpec((1, C), const),
        out_shape=jax.ShapeDtypeStruct((1, C), f32),
        compiler_params=pltpu.CompilerParams(
            vmem_limit_bytes=64 * 1024 * 1024,
        ),
        scratch_shapes=[
            pltpu.VMEM((N, 3 * H), f32),
            pltpu.VMEM((R, H), f32),
            pltpu.SMEM((1, 1), f32),
            pltpu.SMEM((1, 1), f32),
            pltpu.VMEM((1, H), f32),
        ],
    )(x, adj, adj, W1, b1.reshape(1, H), W2, b2.reshape(1, H), Watt,
      batt.reshape(1, 1), Wcls, bcls.reshape(1, C))

    return out.reshape(C)
